# baseline (device time: 19176 ns/iter reference)
import jax
import jax.numpy as jnp
from jax import lax
from jax.experimental import pallas as pl
from jax.experimental.pallas import tpu as pltpu

N_DEV = 4
E_PER = 4
N_TOK = 512
D_IN = 256
D_OUT = 512
N_EXP = 16
CHUNK = N_TOK // N_DEV

SEND_ORDER = (2, 3, 1)


def kernel(x, router_W, route_idx, expert_W):
    def body(x_hbm, rw_ref, idx_ref, ew_hbm, out_ref,
             x_ref, ew_ref, gw_ref, stage, rs_buf, red_buf, p3_buf, ag_buf,
             in_sems, s_rs, r_rs, s_ag, r_ag):
        me = lax.axis_index("i")

        x_dma = pltpu.make_async_copy(x_hbm, x_ref, in_sems.at[0])
        ew_dma = pltpu.make_async_copy(ew_hbm, ew_ref, in_sems.at[1])
        x_dma.start()
        ew_dma.start()

        barrier_sem = pltpu.get_barrier_semaphore()
        for d in range(1, N_DEV):
            pl.semaphore_signal(
                barrier_sem, inc=1,
                device_id=(lax.rem(me + d, N_DEV),),
                device_id_type=pl.DeviceIdType.MESH,
            )

        x_dma.wait()
        xv = x_ref[:, :]
        scores = jnp.dot(xv, rw_ref[:, :], preferred_element_type=jnp.float32)
        s_max = jnp.max(scores, axis=1, keepdims=True)
        p = jnp.exp(scores - s_max)
        probs = p / jnp.sum(p, axis=1, keepdims=True)

        e0 = idx_ref[:, 0:1]
        e1 = idx_ref[:, 1:2]
        iota = lax.broadcasted_iota(jnp.int32, (N_TOK, N_EXP), 1)
        g0 = jnp.sum(jnp.where(iota == e0, probs, 0.0), axis=1, keepdims=True)
        g1 = jnp.sum(jnp.where(iota == e1, probs, 0.0), axis=1, keepdims=True)
        gs = g0 + g1
        gates = []
        for le in range(E_PER):
            e_glob = me * E_PER + le
            gates.append((jnp.where(e0 == e_glob, g0, 0.0)
                          + jnp.where(e1 == e_glob, g1, 0.0)) / gs)
        gw_ref[:, :] = jnp.concatenate(gates, axis=1)

        ew_dma.wait()
        ew = ew_ref[:, :, :].reshape(E_PER * D_IN, D_OUT).astype(jnp.bfloat16)

        def chunk_for(q):
            xc = x_ref[pl.ds(q * CHUNK, CHUNK), :]
            parts = [xc * gw_ref[pl.ds(q * CHUNK, CHUNK), le:le + 1]
                     for le in range(E_PER)]
            return jnp.dot(jnp.concatenate(parts, axis=1).astype(jnp.bfloat16),
                           ew, preferred_element_type=jnp.float32)

        by_slot = {}
        rs_sends = []
        barrier_waited = False
        for d in SEND_ORDER:
            q = lax.rem(me + d, N_DEV)
            stage[d, :, :] = chunk_for(q).astype(jnp.bfloat16)
            if not barrier_waited:
                pl.semaphore_wait(barrier_sem, N_DEV - 1)
                barrier_waited = True
            rdma = pltpu.make_async_remote_copy(
                src_ref=stage.at[d],
                dst_ref=rs_buf.at[N_DEV - d],
                send_sem=s_rs.at[d],
                recv_sem=r_rs.at[N_DEV - d],
                device_id=(q,),
                device_id_type=pl.DeviceIdType.MESH,
            )
            rdma.start()
            rs_sends.append(rdma)
            by_slot[N_DEV - d] = rdma

        mine = chunk_for(me)
        by_slot[3].wait_recv()
        by_slot[1].wait_recv()
        partial = (mine + rs_buf[3, :, :].astype(jnp.float32)
                   + rs_buf[1, :, :].astype(jnp.float32))

        p3_buf[:, :] = partial.astype(jnp.bfloat16)
        diag = pltpu.make_async_remote_copy(
            src_ref=p3_buf,
            dst_ref=ag_buf.at[2],
            send_sem=s_ag.at[2],
            recv_sem=r_ag.at[2],
            device_id=(lax.rem(me + 2, N_DEV),),
            device_id_type=pl.DeviceIdType.MESH,
        )
        diag.start()

        by_slot[2].wait_recv()
        red = partial + rs_buf[2, :, :].astype(jnp.float32)
        out_ref[pl.ds(me * CHUNK, CHUNK), :] = red
        red_buf[:, :] = red.astype(jnp.bfloat16)

        ag_sends = [diag]
        for d in (3, 1):
            q = lax.rem(me + d, N_DEV)
            rdma = pltpu.make_async_remote_copy(
                src_ref=red_buf,
                dst_ref=ag_buf.at[N_DEV - d],
                send_sem=s_ag.at[d],
                recv_sem=r_ag.at[N_DEV - d],
                device_id=(q,),
                device_id_type=pl.DeviceIdType.MESH,
            )
            rdma.start()
            ag_sends.append(rdma)

        ag_by_slot = {2: ag_sends[0], 1: ag_sends[1], 3: ag_sends[2]}
        for k in (3, 1):
            ag_by_slot[k].wait_recv()
            src = lax.rem(me + k, N_DEV)
            out_ref[pl.ds(src * CHUNK, CHUNK), :] = (
                ag_buf[k, :, :].astype(jnp.float32))
        ag_by_slot[2].wait_recv()
        src2 = lax.rem(me + 2, N_DEV)
        out_ref[pl.ds(src2 * CHUNK, CHUNK), :] = (
            ag_buf[2, :, :].astype(jnp.float32)
            + stage[2, :, :].astype(jnp.float32))

        for rdma in rs_sends + ag_sends:
            rdma.wait_send()

    return pl.pallas_call(
        body,
        out_shape=jax.ShapeDtypeStruct((N_TOK, D_OUT), jnp.float32),
        in_specs=[
            pl.BlockSpec(memory_space=pl.ANY),
            pl.BlockSpec(memory_space=pltpu.VMEM),
            pl.BlockSpec(memory_space=pltpu.VMEM),
            pl.BlockSpec(memory_space=pl.ANY),
        ],
        out_specs=pl.BlockSpec(memory_space=pltpu.VMEM),
        scratch_shapes=[
            pltpu.VMEM((N_TOK, D_IN), jnp.float32),
            pltpu.VMEM((E_PER, D_IN, D_OUT), jnp.float32),
            pltpu.VMEM((N_TOK, E_PER), jnp.float32),
            pltpu.VMEM((N_DEV, CHUNK, D_OUT), jnp.bfloat16),
            pltpu.VMEM((N_DEV, CHUNK, D_OUT), jnp.bfloat16),
            pltpu.VMEM((CHUNK, D_OUT), jnp.bfloat16),
            pltpu.VMEM((CHUNK, D_OUT), jnp.bfloat16),
            pltpu.VMEM((N_DEV, CHUNK, D_OUT), jnp.bfloat16),
            pltpu.SemaphoreType.DMA((2,)),
            pltpu.SemaphoreType.DMA((N_DEV,)),
            pltpu.SemaphoreType.DMA((N_DEV,)),
            pltpu.SemaphoreType.DMA((N_DEV,)),
            pltpu.SemaphoreType.DMA((N_DEV,)),
        ],
        compiler_params=pltpu.CompilerParams(collective_id=0),
    )(x, router_W, route_idx, expert_W)


# device time: 18785 ns/iter; 1.0208x vs baseline; 1.0208x over previous
import jax
import jax.numpy as jnp
from jax import lax
from jax.experimental import pallas as pl
from jax.experimental.pallas import tpu as pltpu

N_DEV = 4
E_PER = 4
N_TOK = 512
D_IN = 256
D_OUT = 512
N_EXP = 16
CHUNK = N_TOK // N_DEV

SEND_ORDER = (2, 3, 1)


def kernel(x, router_W, route_idx, expert_W):
    def body(x_ref, rw_ref, idx_ref, ew_ref, out_ref,
             gw_ref, stage, rs_buf, red_buf, p3_buf, ag_buf,
             s_rs, r_rs, s_ag, r_ag):
        me = lax.axis_index("i")

        barrier_sem = pltpu.get_barrier_semaphore()
        for d in range(1, N_DEV):
            pl.semaphore_signal(
                barrier_sem, inc=1,
                device_id=(lax.rem(me + d, N_DEV),),
                device_id_type=pl.DeviceIdType.MESH,
            )

        xv = x_ref[:, :]
        scores = jnp.dot(xv, rw_ref[:, :], preferred_element_type=jnp.float32)
        s_max = jnp.max(scores, axis=1, keepdims=True)
        p = jnp.exp(scores - s_max)
        probs = p / jnp.sum(p, axis=1, keepdims=True)

        e0 = idx_ref[:, 0:1]
        e1 = idx_ref[:, 1:2]
        iota = lax.broadcasted_iota(jnp.int32, (N_TOK, N_EXP), 1)
        g0 = jnp.sum(jnp.where(iota == e0, probs, 0.0), axis=1, keepdims=True)
        g1 = jnp.sum(jnp.where(iota == e1, probs, 0.0), axis=1, keepdims=True)
        gs = g0 + g1
        gates = []
        for le in range(E_PER):
            e_glob = me * E_PER + le
            gates.append((jnp.where(e0 == e_glob, g0, 0.0)
                          + jnp.where(e1 == e_glob, g1, 0.0)) / gs)
        gw_ref[:, :] = jnp.concatenate(gates, axis=1)

        ew = ew_ref[:, :, :].reshape(E_PER * D_IN, D_OUT).astype(jnp.bfloat16)

        def chunk_for(q):
            xc = x_ref[pl.ds(q * CHUNK, CHUNK), :]
            parts = [xc * gw_ref[pl.ds(q * CHUNK, CHUNK), le:le + 1]
                     for le in range(E_PER)]
            return jnp.dot(jnp.concatenate(parts, axis=1).astype(jnp.bfloat16),
                           ew, preferred_element_type=jnp.float32)

        by_slot = {}
        rs_sends = []
        barrier_waited = False
        for d in SEND_ORDER:
            q = lax.rem(me + d, N_DEV)
            stage[d, :, :] = chunk_for(q).astype(jnp.bfloat16)
            if not barrier_waited:
                pl.semaphore_wait(barrier_sem, N_DEV - 1)
                barrier_waited = True
            rdma = pltpu.make_async_remote_copy(
                src_ref=stage.at[d],
                dst_ref=rs_buf.at[N_DEV - d],
                send_sem=s_rs.at[d],
                recv_sem=r_rs.at[N_DEV - d],
                device_id=(q,),
                device_id_type=pl.DeviceIdType.MESH,
            )
            rdma.start()
            rs_sends.append(rdma)
            by_slot[N_DEV - d] = rdma

        mine = chunk_for(me)
        by_slot[3].wait_recv()
        by_slot[1].wait_recv()
        partial = (mine + rs_buf[3, :, :].astype(jnp.float32)
                   + rs_buf[1, :, :].astype(jnp.float32))

        p3_buf[:, :] = partial.astype(jnp.bfloat16)
        diag = pltpu.make_async_remote_copy(
            src_ref=p3_buf,
            dst_ref=ag_buf.at[2],
            send_sem=s_ag.at[2],
            recv_sem=r_ag.at[2],
            device_id=(lax.rem(me + 2, N_DEV),),
            device_id_type=pl.DeviceIdType.MESH,
        )
        diag.start()

        by_slot[2].wait_recv()
        red = partial + rs_buf[2, :, :].astype(jnp.float32)
        out_ref[pl.ds(me * CHUNK, CHUNK), :] = red
        red_buf[:, :] = red.astype(jnp.bfloat16)

        ag_sends = [diag]
        for d in (3, 1):
            q = lax.rem(me + d, N_DEV)
            rdma = pltpu.make_async_remote_copy(
                src_ref=red_buf,
                dst_ref=ag_buf.at[N_DEV - d],
                send_sem=s_ag.at[d],
                recv_sem=r_ag.at[N_DEV - d],
                device_id=(q,),
                device_id_type=pl.DeviceIdType.MESH,
            )
            rdma.start()
            ag_sends.append(rdma)

        ag_by_slot = {2: ag_sends[0], 1: ag_sends[1], 3: ag_sends[2]}
        for k in (3, 1):
            ag_by_slot[k].wait_recv()
            src = lax.rem(me + k, N_DEV)
            out_ref[pl.ds(src * CHUNK, CHUNK), :] = (
                ag_buf[k, :, :].astype(jnp.float32))
        ag_by_slot[2].wait_recv()
        src2 = lax.rem(me + 2, N_DEV)
        out_ref[pl.ds(src2 * CHUNK, CHUNK), :] = (
            ag_buf[2, :, :].astype(jnp.float32)
            + stage[2, :, :].astype(jnp.float32))

        for rdma in rs_sends + ag_sends:
            rdma.wait_send()

    return pl.pallas_call(
        body,
        out_shape=jax.ShapeDtypeStruct((N_TOK, D_OUT), jnp.float32),
        in_specs=[
            pl.BlockSpec(memory_space=pltpu.VMEM),
            pl.BlockSpec(memory_space=pltpu.VMEM),
            pl.BlockSpec(memory_space=pltpu.VMEM),
            pl.BlockSpec(memory_space=pltpu.VMEM),
        ],
        out_specs=pl.BlockSpec(memory_space=pltpu.VMEM),
        scratch_shapes=[
            pltpu.VMEM((N_TOK, E_PER), jnp.float32),
            pltpu.VMEM((N_DEV, CHUNK, D_OUT), jnp.bfloat16),
            pltpu.VMEM((N_DEV, CHUNK, D_OUT), jnp.bfloat16),
            pltpu.VMEM((CHUNK, D_OUT), jnp.bfloat16),
            pltpu.VMEM((CHUNK, D_OUT), jnp.bfloat16),
            pltpu.VMEM((N_DEV, CHUNK, D_OUT), jnp.bfloat16),
            pltpu.SemaphoreType.DMA((N_DEV,)),
            pltpu.SemaphoreType.DMA((N_DEV,)),
            pltpu.SemaphoreType.DMA((N_DEV,)),
            pltpu.SemaphoreType.DMA((N_DEV,)),
        ],
        compiler_params=pltpu.CompilerParams(collective_id=0),
    )(x, router_W, route_idx, expert_W)
